# Initial kernel scaffold; baseline (speedup 1.0000x reference)
import jax
import jax.numpy as jnp
from jax.experimental import pallas as pl


def kernel(query, support, WQ, bQ, WK, bK):
    raise NotImplementedError("write your pallas kernel here")


# speedup vs baseline: 2.1942x; 2.1942x over previous
import dataclasses
import functools

import jax
import jax.numpy as jnp
from jax import lax
from jax.experimental import pallas as pl
from jax.experimental.pallas import tpu as pltpu
from jax.experimental.pallas import tpu_sc as plsc

N_Q = 64
N_S = 8192
H = 64
K_TOP = 32
K_CAND = 64
SB = 128
SBW = N_S // SB
NV = SBW // 16
BIG = 1 << 20
NEG_INF = float("-inf")


def _score_body(qa_ref, qb_ref, sa_ref, sb_ref, wq_ref, bq_ref, wk_ref,
                bk_ref, ones_ref, s_ref, m2_ref, qp_ref, kp_ref):
    dn = (((1,), (1,)), ((), ()))
    mm = functools.partial(lax.dot_general, dimension_numbers=dn,
                           preferred_element_type=jnp.float32)
    bq = bq_ref[...]
    bk = bk_ref[...]
    qa = mm(qa_ref[...], wq_ref[...]) + bq
    qb = mm(qb_ref[...], wq_ref[...]) + bq
    ka = mm(sa_ref[...], wk_ref[...]) + bk
    kb = mm(sb_ref[...], wk_ref[...]) + bk
    qp_ref[...] = jnp.concatenate([qa, qb], axis=1)
    kp_ref[...] = jnp.concatenate([ka, kb], axis=1)
    norm_t = mm(ones_ref[...], ka * ka + kb * kb)
    s = 2.0 * (mm(qa, ka) + mm(qb, kb)) - norm_t
    s_ref[...] = s
    m2_ref[...] = jnp.max(s.reshape(N_Q, SB, SBW), axis=2)


def _scores(qa, qb, sa, sb, WQ, bQ, WK, bK):
    ones = jnp.ones((1, H), dtype=jnp.float32)
    return pl.pallas_call(
        _score_body,
        out_shape=[
            jax.ShapeDtypeStruct((N_Q, N_S), jnp.float32),
            jax.ShapeDtypeStruct((N_Q, SB), jnp.float32),
            jax.ShapeDtypeStruct((N_Q, 2 * H), jnp.float32),
            jax.ShapeDtypeStruct((N_S, 2 * H), jnp.float32),
        ],
    )(qa, qb, sa, sb, WQ, bQ.reshape(1, H), WK, bK.reshape(1, H), ones)


def _sc_compiler_params():
    cp = pltpu.CompilerParams()
    if "needs_layout_passes" in pltpu.CompilerParams.__dataclass_fields__:
        cp = dataclasses.replace(cp, needs_layout_passes=False)
    return cp


def _sc_mesh():
    return plsc.VectorSubcoreMesh(core_axis_name="c", subcore_axis_name="s",
                                  num_cores=2, num_subcores=16)


def _coarse_body(s_hbm, m2_hbm, kp_hbm, idx64_hbm, g_hbm, row_v, m2_v,
                 idxs_v, g_v):
    wid = lax.axis_index("s") * 2 + lax.axis_index("c")
    iota = lax.iota(jnp.int32, 16)

    @pl.loop(0, 2)
    def _row(j):
        r = wid * 2 + j
        pltpu.sync_copy(s_hbm.at[r], row_v)
        pltpu.sync_copy(m2_hbm.at[r], m2_v)

        @pl.loop(0, K_CAND)
        def _extract(i):
            chunks = [m2_v[pl.ds(cc * 16, 16)] for cc in range(SB // 16)]
            h = chunks[0]
            for cc in range(1, SB // 16):
                h = jnp.maximum(h, chunks[cc])
            gmax = jnp.max(h)
            g_star = BIG
            for cc in range(SB // 16):
                cand = jnp.min(jnp.where(chunks[cc] == gmax,
                                         iota + cc * 16, BIG))
                g_star = jnp.minimum(g_star, cand)
            base = g_star * SBW
            pos = BIG
            for t in range(NV):
                vt = row_v[pl.ds(base + t * 16, 16)]
                cand = jnp.min(jnp.where(vt == gmax, iota + t * 16, BIG))
                pos = jnp.minimum(pos, cand)
            el = base + pos
            voff = base + (pos >> 4) * 16
            vm = row_v[pl.ds(voff, 16)]
            row_v[pl.ds(voff, 16)] = jnp.where(iota == (pos & 15), NEG_INF, vm)
            nb = row_v[pl.ds(base, 16)]
            for t in range(1, NV):
                nb = jnp.maximum(nb, row_v[pl.ds(base + t * 16, 16)])
            bm = jnp.max(nb)
            coff = (g_star >> 4) * 16
            mv = m2_v[pl.ds(coff, 16)]
            m2_v[pl.ds(coff, 16)] = jnp.where(iota == (g_star & 15), bm, mv)
            ooff = (i >> 4) * 16
            iv = idxs_v[pl.ds(ooff, 16)]
            idxs_v[pl.ds(ooff, 16)] = jnp.where(iota == (i & 15), el, iv)

        pltpu.sync_copy(kp_hbm.at[idxs_v], g_v)
        pltpu.sync_copy(idxs_v, idx64_hbm.at[r])
        pltpu.sync_copy(g_v, g_hbm.at[pl.ds(r * K_CAND, K_CAND)])


def _coarse_topk(s, m2, kp):
    fn = pl.kernel(
        _coarse_body,
        out_type=[
            jax.ShapeDtypeStruct((N_Q, K_CAND), jnp.int32),
            jax.ShapeDtypeStruct((N_Q * K_CAND, 2 * H), jnp.float32),
        ],
        mesh=_sc_mesh(),
        scratch_types=[
            pltpu.VMEM((N_S,), jnp.float32),
            pltpu.VMEM((SB,), jnp.float32),
            pltpu.VMEM((K_CAND,), jnp.int32),
            pltpu.VMEM((K_CAND, 2 * H), jnp.float32),
        ],
        compiler_params=_sc_compiler_params(),
    )
    return fn(s, m2, kp)


def _rescore_body(qp_ref, g_ref, e_ref):
    for r in range(N_Q):
        g = g_ref[pl.ds(r * K_CAND, K_CAND), :]
        diff = qp_ref[r:r + 1, :] - g
        s = diff * diff
        a = s[:, 0:8]
        for j in range(1, 16):
            a = a + s[:, j * 8:(j + 1) * 8]
        a = a[:, :4] + a[:, 4:]
        a = a[:, :2] + a[:, 2:]
        d2 = a[:, 0:1] + a[:, 1:2]
        dd = jnp.sqrt(d2)
        e_ref[pl.ds(r * K_CAND, K_CAND), :] = -(dd * dd) * 0.125


def _rescore(qp, g):
    return pl.pallas_call(
        _rescore_body,
        out_shape=jax.ShapeDtypeStruct((N_Q * K_CAND, 1), jnp.float32),
    )(qp, g)


def _final_body(e_hbm, idx64_hbm, idx_hbm, w_hbm, ev, iv, vals_v, idxs_v):
    wid = lax.axis_index("s") * 2 + lax.axis_index("c")
    iota = lax.iota(jnp.int32, 16)
    nch = K_CAND // 16

    @pl.loop(0, 2)
    def _row(j):
        r = wid * 2 + j
        pltpu.sync_copy(e_hbm.at[r], ev)
        pltpu.sync_copy(idx64_hbm.at[r], iv)

        @pl.loop(0, K_TOP)
        def _extract(i):
            vts = [ev[pl.ds(t * 16, 16)] for t in range(nch)]
            its = [iv[pl.ds(t * 16, 16)] for t in range(nch)]
            h = vts[0]
            for t in range(1, nch):
                h = jnp.maximum(h, vts[t])
            gmax = jnp.max(h)
            el = BIG
            for t in range(nch):
                el = jnp.minimum(el, jnp.min(jnp.where(vts[t] == gmax,
                                                       its[t], BIG)))
            for t in range(nch):
                hit = (vts[t] == gmax) & (its[t] == el)
                ev[pl.ds(t * 16, 16)] = jnp.where(hit, NEG_INF, vts[t])
            ooff = (i >> 4) * 16
            vv = vals_v[pl.ds(ooff, 16)]
            vals_v[pl.ds(ooff, 16)] = jnp.where(iota == (i & 15), gmax, vv)
            ivv = idxs_v[pl.ds(ooff, 16)]
            idxs_v[pl.ds(ooff, 16)] = jnp.where(iota == (i & 15), el, ivv)

        v0 = vals_v[pl.ds(0, 16)]
        v1 = vals_v[pl.ds(16, 16)]
        mx = jnp.max(v0)
        e0 = jnp.exp(v0 - mx)
        e1 = jnp.exp(v1 - mx)
        tot = jnp.sum(e0) + jnp.sum(e1)
        vals_v[pl.ds(0, 16)] = e0 / tot
        vals_v[pl.ds(16, 16)] = e1 / tot
        pltpu.sync_copy(idxs_v, idx_hbm.at[r])
        pltpu.sync_copy(vals_v, w_hbm.at[r])


def _final(e, idx64):
    fn = pl.kernel(
        _final_body,
        out_type=[
            jax.ShapeDtypeStruct((N_Q, K_TOP), jnp.int32),
            jax.ShapeDtypeStruct((N_Q, K_TOP), jnp.float32),
        ],
        mesh=_sc_mesh(),
        scratch_types=[
            pltpu.VMEM((K_CAND,), jnp.float32),
            pltpu.VMEM((K_CAND,), jnp.int32),
            pltpu.VMEM((K_TOP,), jnp.float32),
            pltpu.VMEM((K_TOP,), jnp.int32),
        ],
        compiler_params=_sc_compiler_params(),
    )
    return fn(e, idx64)


def kernel(query, support, WQ, bQ, WK, bK):
    qa = query[:, 0, :]
    qb = query[:, 1, :]
    sa = support[:, 0, :]
    sb = support[:, 1, :]
    s, m2, qp, kp = _scores(qa, qb, sa, sb, WQ, bQ, WK, bK)
    idx64, g = _coarse_topk(s, m2, kp)
    e = _rescore(qp, g)
    idx, w = _final(e.reshape(N_Q, K_CAND), idx64)
    return idx, w


# speedup vs baseline: 2.7211x; 1.2402x over previous
import dataclasses
import functools

import jax
import jax.numpy as jnp
from jax import lax
from jax.experimental import pallas as pl
from jax.experimental.pallas import tpu as pltpu
from jax.experimental.pallas import tpu_sc as plsc

N_Q = 64
N_S = 8192
H = 64
K_TOP = 32
K_CAND = 48
SB = 128
SBW = N_S // SB
NV = SBW // 16
BIG = 1 << 20
NEG_INF = float("-inf")


def _score_body(qa_ref, qb_ref, sa_ref, sb_ref, wq_ref, bq_ref, wk_ref,
                bk_ref, ones_ref, s_ref, m2_ref, qp_ref, kp_ref):
    dn = (((1,), (1,)), ((), ()))
    mm = functools.partial(lax.dot_general, dimension_numbers=dn,
                           preferred_element_type=jnp.float32)
    bq = bq_ref[...]
    bk = bk_ref[...]
    qa = mm(qa_ref[...], wq_ref[...]) + bq
    qb = mm(qb_ref[...], wq_ref[...]) + bq
    ka = mm(sa_ref[...], wk_ref[...]) + bk
    kb = mm(sb_ref[...], wk_ref[...]) + bk
    qp_ref[...] = jnp.concatenate([qa, qb], axis=1)
    kp_ref[...] = jnp.concatenate([ka, kb], axis=1)
    norm_t = mm(ones_ref[...], ka * ka + kb * kb)
    s = 2.0 * (mm(qa, ka) + mm(qb, kb)) - norm_t
    s_ref[...] = s
    m2_ref[...] = jnp.max(s.reshape(N_Q, SB, SBW), axis=2)


def _scores(qa, qb, sa, sb, WQ, bQ, WK, bK):
    ones = jnp.ones((1, H), dtype=jnp.float32)
    return pl.pallas_call(
        _score_body,
        out_shape=[
            jax.ShapeDtypeStruct((N_Q, N_S), jnp.float32),
            jax.ShapeDtypeStruct((N_Q, SB), jnp.float32),
            jax.ShapeDtypeStruct((N_Q, 2 * H), jnp.float32),
            jax.ShapeDtypeStruct((N_S, 2 * H), jnp.float32),
        ],
    )(qa, qb, sa, sb, WQ, bQ.reshape(1, H), WK, bK.reshape(1, H), ones)


def _sc_compiler_params():
    cp = pltpu.CompilerParams()
    if "needs_layout_passes" in pltpu.CompilerParams.__dataclass_fields__:
        cp = dataclasses.replace(cp, needs_layout_passes=False)
    return cp


def _coarse_body(s_hbm, m2_hbm, kp_hbm, idx64_hbm, g_hbm, row_v, m2_v,
                 idxs_v, g_v):
    wid = lax.axis_index("s") * 2 + lax.axis_index("c")
    iota = lax.iota(jnp.int32, 16)

    @pl.loop(0, 2)
    def _row(j):
        r = wid * 2 + j
        pltpu.sync_copy(s_hbm.at[r], row_v)
        pltpu.sync_copy(m2_hbm.at[r], m2_v)

        @pl.loop(0, K_CAND)
        def _extract(i):
            chunks = [m2_v[pl.ds(cc * 16, 16)] for cc in range(SB // 16)]
            h = chunks[0]
            for cc in range(1, SB // 16):
                h = jnp.maximum(h, chunks[cc])
            gmax = jnp.max(h)
            g_star = BIG
            for cc in range(SB // 16):
                cand = jnp.min(jnp.where(chunks[cc] == gmax,
                                         iota + cc * 16, BIG))
                g_star = jnp.minimum(g_star, cand)
            base = g_star * SBW
            pos = BIG
            for t in range(NV):
                vt = row_v[pl.ds(base + t * 16, 16)]
                cand = jnp.min(jnp.where(vt == gmax, iota + t * 16, BIG))
                pos = jnp.minimum(pos, cand)
            el = base + pos
            voff = base + (pos >> 4) * 16
            vm = row_v[pl.ds(voff, 16)]
            row_v[pl.ds(voff, 16)] = jnp.where(iota == (pos & 15), NEG_INF, vm)
            nb = row_v[pl.ds(base, 16)]
            for t in range(1, NV):
                nb = jnp.maximum(nb, row_v[pl.ds(base + t * 16, 16)])
            bm = jnp.max(nb)
            coff = (g_star >> 4) * 16
            mv = m2_v[pl.ds(coff, 16)]
            m2_v[pl.ds(coff, 16)] = jnp.where(iota == (g_star & 15), bm, mv)
            ooff = (i >> 4) * 16
            iv = idxs_v[pl.ds(ooff, 16)]
            idxs_v[pl.ds(ooff, 16)] = jnp.where(iota == (i & 15), el, iv)

        pltpu.sync_copy(kp_hbm.at[idxs_v], g_v)
        pltpu.sync_copy(idxs_v, idx64_hbm.at[r])
        pltpu.sync_copy(g_v, g_hbm.at[pl.ds(r * K_CAND, K_CAND)])


def _coarse_topk(s, m2, kp):
    mesh = plsc.VectorSubcoreMesh(core_axis_name="c", subcore_axis_name="s",
                                  num_cores=2, num_subcores=16)
    fn = pl.kernel(
        _coarse_body,
        out_type=[
            jax.ShapeDtypeStruct((N_Q, K_CAND), jnp.int32),
            jax.ShapeDtypeStruct((N_Q * K_CAND, 2 * H), jnp.float32),
        ],
        mesh=mesh,
        scratch_types=[
            pltpu.VMEM((N_S,), jnp.float32),
            pltpu.VMEM((SB,), jnp.float32),
            pltpu.VMEM((K_CAND,), jnp.int32),
            pltpu.VMEM((K_CAND, 2 * H), jnp.float32),
        ],
        compiler_params=_sc_compiler_params(),
    )
    return fn(s, m2, kp)


def _rescore_body(qx_ref, g_ref, idx64_ref, idx_ref, w_ref):
    diff = qx_ref[...] - g_ref[...]
    s = diff * diff
    a = s[:, 0:8]
    for j in range(1, 16):
        a = a + s[:, j * 8:(j + 1) * 8]
    a = a[:, :4] + a[:, 4:]
    a = a[:, :2] + a[:, 2:]
    d2 = a[:, 0:1] + a[:, 1:2]
    dd = jnp.sqrt(d2)
    sim = -(dd * dd) * 0.125
    e = sim.reshape(N_Q, K_CAND)
    ix = idx64_ref[...]
    vals_cols = []
    idx_cols = []
    for _ in range(K_TOP):
        rowmax = jnp.max(e, axis=1, keepdims=True)
        hit = e == rowmax
        el = jnp.min(jnp.where(hit, ix, BIG), axis=1, keepdims=True)
        e = jnp.where(hit & (ix == el), NEG_INF, e)
        vals_cols.append(rowmax)
        idx_cols.append(el)
    vals = jnp.concatenate(vals_cols, axis=1)
    idx_ref[...] = jnp.concatenate(idx_cols, axis=1)
    ex = jnp.exp(vals - vals[:, 0:1])
    w_ref[...] = ex / jnp.sum(ex, axis=1, keepdims=True)


def _rescore_body(qx_ref, g_ref, e_ref):
    diff = qx_ref[...] - g_ref[...]
    s = diff * diff
    a = s[:, 0:8]
    for j in range(1, 16):
        a = a + s[:, j * 8:(j + 1) * 8]
    a = a[:, :4] + a[:, 4:]
    a = a[:, :2] + a[:, 2:]
    d2 = a[:, 0:1] + a[:, 1:2]
    dd = jnp.sqrt(d2)
    sim = -(dd * dd) * 0.125
    e_ref[...] = sim.reshape(N_Q, K_CAND)


def _rescore(qx, g):
    return pl.pallas_call(
        _rescore_body,
        out_shape=jax.ShapeDtypeStruct((N_Q, K_CAND), jnp.float32),
    )(qx, g)


def _final_body(e_hbm, idx64_hbm, idx_hbm, w_hbm, ev, iv, vals_v, idxs_v):
    wid = lax.axis_index("s") * 2 + lax.axis_index("c")
    iota = lax.iota(jnp.int32, 16)
    nch = K_CAND // 16

    @pl.loop(0, 2)
    def _row(j):
        r = wid * 2 + j
        pltpu.sync_copy(e_hbm.at[r], ev)
        pltpu.sync_copy(idx64_hbm.at[r], iv)

        @pl.loop(0, K_TOP)
        def _extract(i):
            vts = [ev[pl.ds(t * 16, 16)] for t in range(nch)]
            its = [iv[pl.ds(t * 16, 16)] for t in range(nch)]
            h = vts[0]
            for t in range(1, nch):
                h = jnp.maximum(h, vts[t])
            gmax = jnp.max(h)
            el = BIG
            for t in range(nch):
                el = jnp.minimum(el, jnp.min(jnp.where(vts[t] == gmax,
                                                       its[t], BIG)))
            for t in range(nch):
                hit = (vts[t] == gmax) & (its[t] == el)
                ev[pl.ds(t * 16, 16)] = jnp.where(hit, NEG_INF, vts[t])
            ooff = (i >> 4) * 16
            vv = vals_v[pl.ds(ooff, 16)]
            vals_v[pl.ds(ooff, 16)] = jnp.where(iota == (i & 15), gmax, vv)
            ivv = idxs_v[pl.ds(ooff, 16)]
            idxs_v[pl.ds(ooff, 16)] = jnp.where(iota == (i & 15), el, ivv)

        v0 = vals_v[pl.ds(0, 16)]
        v1 = vals_v[pl.ds(16, 16)]
        mx = jnp.max(v0)
        e0 = jnp.exp(v0 - mx)
        e1 = jnp.exp(v1 - mx)
        tot = jnp.sum(e0) + jnp.sum(e1)
        vals_v[pl.ds(0, 16)] = e0 / tot
        vals_v[pl.ds(16, 16)] = e1 / tot
        pltpu.sync_copy(idxs_v, idx_hbm.at[r])
        pltpu.sync_copy(vals_v, w_hbm.at[r])


def _final(e, idx64):
    mesh = plsc.VectorSubcoreMesh(core_axis_name="c", subcore_axis_name="s",
                                  num_cores=2, num_subcores=16)
    fn = pl.kernel(
        _final_body,
        out_type=[
            jax.ShapeDtypeStruct((N_Q, K_TOP), jnp.int32),
            jax.ShapeDtypeStruct((N_Q, K_TOP), jnp.float32),
        ],
        mesh=mesh,
        scratch_types=[
            pltpu.VMEM((K_CAND,), jnp.float32),
            pltpu.VMEM((K_CAND,), jnp.int32),
            pltpu.VMEM((K_TOP,), jnp.float32),
            pltpu.VMEM((K_TOP,), jnp.int32),
        ],
        compiler_params=_sc_compiler_params(),
    )
    return fn(e, idx64)


def kernel(query, support, WQ, bQ, WK, bK):
    qa = query[:, 0, :]
    qb = query[:, 1, :]
    sa = support[:, 0, :]
    sb = support[:, 1, :]
    s, m2, qp, kp = _scores(qa, qb, sa, sb, WQ, bQ, WK, bK)
    idx64, g = _coarse_topk(s, m2, kp)
    qx = jnp.broadcast_to(qp[:, None, :], (N_Q, K_CAND, 2 * H))
    qx = qx.reshape(N_Q * K_CAND, 2 * H)
    e = _rescore(qx, g)
    idx, w = _final(e, idx64)
    return idx, w
